# Initial kernel scaffold; baseline (speedup 1.0000x reference)
import jax
import jax.numpy as jnp
from jax.experimental import pallas as pl


def kernel(x_dict, adj_dict, W_rel, w_self, bias, w_query, w_keys, w_att):
    raise NotImplementedError("write your pallas kernel here")


# speedup vs baseline: 1.3696x; 1.3696x over previous
import functools

import jax
import jax.numpy as jnp
from jax.experimental import pallas as pl
from jax.experimental.pallas import tpu as pltpu


def _matmul_body(x_ref, w_ref, o_ref):
    o_ref[...] = jnp.dot(x_ref[...], w_ref[...],
                         preferred_element_type=jnp.float32)


def _fused_body(adj_ref, x_ref, hrel_ref, wself_ref, wq_ref, wk_ref,
                watt_ref, bias_ref, o_ref):
    T = wq_ref.shape[1]
    self_ft = jnp.dot(x_ref[...], wself_ref[...],
                      preferred_element_type=jnp.float32)
    nb = jnp.dot(adj_ref[...], hrel_ref[...],
                 preferred_element_type=jnp.float32)

    u_k = jnp.dot(wk_ref[...], watt_ref[:T, :],
                  preferred_element_type=jnp.float32)
    u_q = jnp.dot(wq_ref[...], watt_ref[T:, :],
                  preferred_element_type=jnp.float32)

    s_q = jnp.dot(self_ft, u_q, preferred_element_type=jnp.float32)
    v0 = jnp.dot(self_ft, u_k, preferred_element_type=jnp.float32) + s_q
    v1 = jnp.dot(nb, u_k, preferred_element_type=jnp.float32) + s_q

    e0 = jnp.where(v0 > 0, v0, jnp.exp(jnp.minimum(v0, 0.0)) - 1.0)
    e1 = jnp.where(v1 > 0, v1, jnp.exp(jnp.minimum(v1, 0.0)) - 1.0)

    m = jnp.maximum(e0, e1)
    z0 = jnp.exp(e0 - m)
    z1 = jnp.exp(e1 - m)
    inv = 1.0 / (z0 + z1)
    a0 = z0 * inv
    a1 = z1 * inv

    o_ref[...] = self_ft * a0 + nb * a1 + bias_ref[...]


@jax.jit
def kernel(x_dict, adj_dict, W_rel, w_self, bias, w_query, w_keys, w_att):
    N, DIN = x_dict.shape
    DOUT = W_rel.shape[1]
    T2 = w_att.shape[0]

    BA = 1000
    hrel = pl.pallas_call(
        _matmul_body,
        grid=(N // BA,),
        in_specs=[
            pl.BlockSpec((BA, DIN), lambda i: (i, 0)),
            pl.BlockSpec((DIN, DOUT), lambda i: (0, 0)),
        ],
        out_specs=pl.BlockSpec((BA, DOUT), lambda i: (i, 0)),
        out_shape=jax.ShapeDtypeStruct((N, DOUT), jnp.float32),
        compiler_params=pltpu.CompilerParams(
            dimension_semantics=("arbitrary",)),
    )(x_dict, W_rel)

    BN = 400
    out = pl.pallas_call(
        _fused_body,
        grid=(N // BN,),
        in_specs=[
            pl.BlockSpec((BN, N), lambda i: (i, 0)),
            pl.BlockSpec((BN, DIN), lambda i: (i, 0)),
            pl.BlockSpec((N, DOUT), lambda i: (0, 0)),
            pl.BlockSpec((DIN, DOUT), lambda i: (0, 0)),
            pl.BlockSpec(w_query.shape, lambda i: (0, 0)),
            pl.BlockSpec(w_keys.shape, lambda i: (0, 0)),
            pl.BlockSpec((T2, 1), lambda i: (0, 0)),
            pl.BlockSpec((1, DOUT), lambda i: (0, 0)),
        ],
        out_specs=pl.BlockSpec((BN, DOUT), lambda i: (i, 0)),
        out_shape=jax.ShapeDtypeStruct((N, DOUT), jnp.float32),
        compiler_params=pltpu.CompilerParams(
            dimension_semantics=("arbitrary",)),
    )(adj_dict, x_dict, hrel, w_self, w_query, w_keys, w_att, bias)
    return out
